# Initial kernel scaffold; baseline (speedup 1.0000x reference)
import functools

import jax
import jax.numpy as jnp
from jax import lax
from jax.experimental import pallas as pl
from jax.experimental.pallas import tpu as pltpu
from jax.experimental.pallas import tpu_sc as plsc

NC = 2
NS = 16
NW = NC * NS
K = 80


def _sc_aggregate(y, src, dst):
    n, d = y.shape
    e = src.shape[0]
    e_per_w = e // NW
    chunks = e_per_w // K
    rows_per_sub = n // NS

    mesh = plsc.VectorSubcoreMesh(core_axis_name="c", subcore_axis_name="s")

    @functools.partial(
        pl.kernel,
        out_type=jax.ShapeDtypeStruct((NC, n, d), jnp.float32),
        mesh=mesh,
        scratch_types=[
            pltpu.VMEM((K,), jnp.int32),
            pltpu.VMEM((K,), jnp.int32),
            pltpu.VMEM((K, d), jnp.float32),
            pltpu.VMEM_SHARED((n, d), jnp.float32),
            pltpu.SemaphoreType.DMA,
        ],
    )
    def k(y_hbm, src_hbm, dst_hbm, out_hbm, src_v, dst_v, rows_v, z_sh, sem):
        cid = lax.axis_index("c")
        sid = lax.axis_index("s")
        wid = cid * NS + sid
        r0 = sid * rows_per_sub
        pltpu.sync_copy(y_hbm.at[pl.ds(r0, rows_per_sub)],
                        z_sh.at[pl.ds(r0, rows_per_sub)])
        plsc.subcore_barrier()
        base = wid * e_per_w

        @pl.loop(0, chunks)
        def _(i):
            off = base + i * K
            pltpu.sync_copy(src_hbm.at[pl.ds(off, K)], src_v)
            pltpu.sync_copy(dst_hbm.at[pl.ds(off, K)], dst_v)
            pltpu.async_copy(y_hbm.at[src_v], rows_v, sem).wait()
            pltpu.sync_copy(rows_v, z_sh.at[dst_v], add=True)

        plsc.subcore_barrier()
        pltpu.sync_copy(z_sh.at[pl.ds(r0, rows_per_sub)],
                        out_hbm.at[cid, pl.ds(r0, rows_per_sub)])

    return k(y, src, dst)


def _prescale_body(degp_ref, x_ref, dinv_ref, y1_ref):
    dp = degp_ref[...]
    deg = dp[0, :, 0] + dp[1, :, 0] - 1.0
    dinv = lax.rsqrt(deg)
    dinv_ref[...] = dinv[:, None]
    y1_ref[...] = x_ref[...] * dinv[:, None]


def _tc_prescale(degp, x):
    n = x.shape[0]
    bn = 1000
    g = n // bn
    return pl.pallas_call(
        _prescale_body,
        grid=(g,),
        in_specs=[
            pl.BlockSpec((NC, bn, 16), lambda i: (0, i, 0)),
            pl.BlockSpec((bn, 128), lambda i: (i, 0)),
        ],
        out_specs=[
            pl.BlockSpec((bn, 1), lambda i: (i, 0)),
            pl.BlockSpec((bn, 128), lambda i: (i, 0)),
        ],
        out_shape=[
            jax.ShapeDtypeStruct((n, 1), jnp.float32),
            jax.ShapeDtypeStruct((n, 128), jnp.float32),
        ],
    )(degp, x)


def _mid_body(z1p_ref, y1_ref, dinv_ref, w1_ref, b1_ref, w2_ref, y2_ref):
    dinv = dinv_ref[...]
    a = (z1p_ref[0] + z1p_ref[1] - y1_ref[...]) * dinv
    pre = jnp.dot(a, w1_ref[...], preferred_element_type=jnp.float32)
    pre = pre + b1_ref[...][None, :]
    h = jnp.where(pre > 0, pre, 0.01 * pre)
    hw = jnp.dot(h, w2_ref[...], preferred_element_type=jnp.float32)
    y2_ref[...] = hw * dinv


def _tc_mid(z1p, y1, dinv, W1, b1, W2):
    n = y1.shape[0]
    bn = 1000
    g = n // bn
    dh = W1.shape[1]
    return pl.pallas_call(
        _mid_body,
        grid=(g,),
        in_specs=[
            pl.BlockSpec((NC, bn, 128), lambda i: (0, i, 0)),
            pl.BlockSpec((bn, 128), lambda i: (i, 0)),
            pl.BlockSpec((bn, 1), lambda i: (i, 0)),
            pl.BlockSpec((128, dh), lambda i: (0, 0)),
            pl.BlockSpec((dh,), lambda i: (0,)),
            pl.BlockSpec((dh, 128), lambda i: (0, 0)),
        ],
        out_specs=pl.BlockSpec((bn, 128), lambda i: (i, 0)),
        out_shape=jax.ShapeDtypeStruct((n, 128), jnp.float32),
    )(z1p, y1, dinv, W1, b1, W2)


def _final_body(z2p_ref, y2_ref, dinv_ref, b2_ref, batch_ref, out_ref,
                sums_ref, cnt_ref, ng):
    i = pl.program_id(0)
    bn = y2_ref.shape[0]
    ne = (z2p_ref[0] + z2p_ref[1] - y2_ref[...]) * dinv_ref[...]
    ne = ne + b2_ref[...][None, :]
    seg = batch_ref[...]
    oh = (seg == lax.broadcasted_iota(jnp.int32, (bn, ng), 1))
    oh = oh.astype(jnp.float32)
    dn = (((0,), (0,)), ((), ()))
    sums = lax.dot_general(oh, ne, dn, preferred_element_type=jnp.float32)
    cnt = lax.dot_general(oh, jnp.ones((bn, 1), jnp.float32), dn,
                          preferred_element_type=jnp.float32)

    @pl.when(i == 0)
    def _():
        sums_ref[...] = jnp.zeros_like(sums_ref)
        cnt_ref[...] = jnp.zeros_like(cnt_ref)

    sums_ref[...] += sums
    cnt_ref[...] += cnt

    @pl.when(i == pl.num_programs(0) - 1)
    def _():
        out_ref[...] = sums_ref[...] / jnp.maximum(cnt_ref[...], 1.0)


def _tc_final(z2p, y2, dinv, b2, batch2d, ng):
    n = y2.shape[0]
    bn = 1000
    g = n // bn
    return pl.pallas_call(
        functools.partial(_final_body, ng=ng),
        grid=(g,),
        in_specs=[
            pl.BlockSpec((NC, bn, 128), lambda i: (0, i, 0)),
            pl.BlockSpec((bn, 128), lambda i: (i, 0)),
            pl.BlockSpec((bn, 1), lambda i: (i, 0)),
            pl.BlockSpec((128,), lambda i: (0,)),
            pl.BlockSpec((bn, 1), lambda i: (i, 0)),
        ],
        out_specs=pl.BlockSpec((ng, 128), lambda i: (0, 0)),
        out_shape=jax.ShapeDtypeStruct((ng, 128), jnp.float32),
        scratch_shapes=[
            pltpu.VMEM((ng, 128), jnp.float32),
            pltpu.VMEM((ng, 1), jnp.float32),
        ],
    )(z2p, y2, dinv, b2, batch2d)


def kernel(x, edge_index, batch, W1, b1, W2, b2):
    n = x.shape[0]
    ng = 64
    src = edge_index[0].astype(jnp.int32)
    dst = edge_index[1].astype(jnp.int32)
    batch2d = batch.astype(jnp.int32).reshape(n, 1)
    ones16 = jnp.ones((n, 16), jnp.float32)

    degp = _sc_aggregate(ones16, src, dst)
    dinv, y1 = _tc_prescale(degp, x)
    z1p = _sc_aggregate(y1, src, dst)
    y2 = _tc_mid(z1p, y1, dinv, W1, b1, W2)
    z2p = _sc_aggregate(y2, src, dst)
    return _tc_final(z2p, y2, dinv, b2, batch2d, ng)


# speedup vs baseline: 15.9729x; 15.9729x over previous
import dataclasses
import functools

import jax
import jax.numpy as jnp
from jax import lax
from jax.experimental import pallas as pl
from jax.experimental.pallas import tpu as pltpu
from jax.experimental.pallas import tpu_sc as plsc

def _sc_compiler_params():
    cp = pltpu.CompilerParams()
    if "needs_layout_passes" in pltpu.CompilerParams.__dataclass_fields__:
        cp = dataclasses.replace(cp, needs_layout_passes=False)
    return cp


NC = 2
NS = 16
NW = NC * NS
K = 80


def _sc_aggregate(y, src, dst):
    n, d = y.shape
    e = src.shape[0]
    e_per_w = e // NW
    chunks = e_per_w // K
    rps = (n // NS) // 8 * 8
    rlast = n - (NS - 1) * rps

    mesh = plsc.VectorSubcoreMesh(core_axis_name="c", subcore_axis_name="s")

    @functools.partial(
        pl.kernel,
        out_type=jax.ShapeDtypeStruct((NC, n, d), jnp.float32),
        mesh=mesh,
        scratch_types=[
            pltpu.VMEM((K,), jnp.int32),
            pltpu.VMEM((K,), jnp.int32),
            pltpu.VMEM((K, d), jnp.float32),
            pltpu.VMEM_SHARED((n, d), jnp.float32),
            pltpu.SemaphoreType.DMA,
        ],
    )
    def k(y_hbm, src_hbm, dst_hbm, out_hbm, src_v, dst_v, rows_v, z_sh, sem):
        cid = lax.axis_index("c")
        sid = lax.axis_index("s")
        wid = cid * NS + sid
        r0 = sid * rps

        @pl.when(sid < NS - 1)
        def _():
            pltpu.sync_copy(y_hbm.at[pl.ds(r0, rps)],
                            z_sh.at[pl.ds(r0, rps)])

        @pl.when(sid == NS - 1)
        def _():
            pltpu.sync_copy(y_hbm.at[pl.ds((NS - 1) * rps, rlast)],
                            z_sh.at[pl.ds((NS - 1) * rps, rlast)])

        plsc.subcore_barrier()
        base = wid * e_per_w

        @pl.loop(0, chunks)
        def _(i):
            off = base + i * K
            pltpu.sync_copy(src_hbm.at[pl.ds(off, K)], src_v)
            pltpu.sync_copy(dst_hbm.at[pl.ds(off, K)], dst_v)
            pltpu.async_copy(y_hbm.at[src_v], rows_v, sem).wait()
            pltpu.sync_copy(rows_v, z_sh.at[dst_v], add=True)

        plsc.subcore_barrier()

        @pl.when(sid < NS - 1)
        def _():
            pltpu.sync_copy(z_sh.at[pl.ds(r0, rps)],
                            out_hbm.at[cid, pl.ds(r0, rps)])

        @pl.when(sid == NS - 1)
        def _():
            pltpu.sync_copy(z_sh.at[pl.ds((NS - 1) * rps, rlast)],
                            out_hbm.at[cid, pl.ds((NS - 1) * rps, rlast)])

    return k(y, src, dst)


def _sc_degree(dst, n):
    e = dst.shape[0]
    e_per_w = e // NW

    mesh = plsc.VectorSubcoreMesh(core_axis_name="c", subcore_axis_name="s")

    @functools.partial(
        pl.kernel,
        out_type=jax.ShapeDtypeStruct((NW * n,), jnp.float32),
        mesh=mesh,
        scratch_types=[
            pltpu.VMEM((e_per_w,), jnp.int32),
            pltpu.VMEM((n,), jnp.float32),
        ],
        compiler_params=_sc_compiler_params(),
    )
    def k(dst_hbm, out_hbm, dst_v, deg_v):
        cid = lax.axis_index("c")
        sid = lax.axis_index("s")
        wid = cid * NS + sid
        pltpu.sync_copy(dst_hbm.at[pl.ds(wid * e_per_w, e_per_w)], dst_v)

        @pl.loop(0, n, step=16)
        def _(i):
            deg_v[pl.ds(i, 16)] = jnp.zeros((16,), jnp.float32)

        ones = jnp.ones((16,), jnp.float32)

        @pl.loop(0, e_per_w, step=16)
        def _(i):
            plsc.addupdate_scatter(deg_v, [dst_v[pl.ds(i, 16)]], ones)

        pltpu.sync_copy(deg_v, out_hbm.at[pl.ds(wid * n, n)])

    return k(dst)


def _prescale_body(degp_ref, x_ref, dinv_ref, y1_ref):
    deg = jnp.sum(degp_ref[...], axis=1, keepdims=True) + 1.0
    dinv = lax.rsqrt(deg)
    dinv_ref[...] = dinv
    y1_ref[...] = x_ref[...] * dinv


def _tc_prescale(degp, x):
    n = x.shape[0]
    bn = 1000
    g = n // bn
    return pl.pallas_call(
        _prescale_body,
        grid=(g,),
        in_specs=[
            pl.BlockSpec((bn, NW), lambda i: (i, 0)),
            pl.BlockSpec((bn, 128), lambda i: (i, 0)),
        ],
        out_specs=[
            pl.BlockSpec((bn, 1), lambda i: (i, 0)),
            pl.BlockSpec((bn, 128), lambda i: (i, 0)),
        ],
        out_shape=[
            jax.ShapeDtypeStruct((n, 1), jnp.float32),
            jax.ShapeDtypeStruct((n, 128), jnp.float32),
        ],
    )(degp, x)


def _mid_body(z1p_ref, y1_ref, dinv_ref, w1_ref, b1_ref, w2_ref, y2_ref):
    dinv = dinv_ref[...]
    a = (z1p_ref[0] + z1p_ref[1] - y1_ref[...]) * dinv
    pre = jnp.dot(a, w1_ref[...], preferred_element_type=jnp.float32)
    pre = pre + b1_ref[...][None, :]
    h = jnp.where(pre > 0, pre, 0.01 * pre)
    hw = jnp.dot(h, w2_ref[...], preferred_element_type=jnp.float32)
    y2_ref[...] = hw * dinv


def _tc_mid(z1p, y1, dinv, W1, b1, W2):
    n = y1.shape[0]
    bn = 1000
    g = n // bn
    dh = W1.shape[1]
    return pl.pallas_call(
        _mid_body,
        grid=(g,),
        in_specs=[
            pl.BlockSpec((NC, bn, 128), lambda i: (0, i, 0)),
            pl.BlockSpec((bn, 128), lambda i: (i, 0)),
            pl.BlockSpec((bn, 1), lambda i: (i, 0)),
            pl.BlockSpec((128, dh), lambda i: (0, 0)),
            pl.BlockSpec((dh,), lambda i: (0,)),
            pl.BlockSpec((dh, 128), lambda i: (0, 0)),
        ],
        out_specs=pl.BlockSpec((bn, 128), lambda i: (i, 0)),
        out_shape=jax.ShapeDtypeStruct((n, 128), jnp.float32),
    )(z1p, y1, dinv, W1, b1, W2)


def _final_body(z2p_ref, y2_ref, dinv_ref, b2_ref, batch_ref, out_ref,
                sums_ref, cnt_ref, ng):
    i = pl.program_id(0)
    bn = y2_ref.shape[0]
    ne = (z2p_ref[0] + z2p_ref[1] - y2_ref[...]) * dinv_ref[...]
    ne = ne + b2_ref[...][None, :]
    seg = batch_ref[...]
    oh = (seg == lax.broadcasted_iota(jnp.int32, (bn, ng), 1))
    oh = oh.astype(jnp.float32)
    dn = (((0,), (0,)), ((), ()))
    sums = lax.dot_general(oh, ne, dn, preferred_element_type=jnp.float32)
    cnt = lax.dot_general(oh, jnp.ones((bn, 1), jnp.float32), dn,
                          preferred_element_type=jnp.float32)

    @pl.when(i == 0)
    def _():
        sums_ref[...] = jnp.zeros_like(sums_ref)
        cnt_ref[...] = jnp.zeros_like(cnt_ref)

    sums_ref[...] += sums
    cnt_ref[...] += cnt

    @pl.when(i == pl.num_programs(0) - 1)
    def _():
        out_ref[...] = sums_ref[...] / jnp.maximum(cnt_ref[...], 1.0)


def _tc_final(z2p, y2, dinv, b2, batch2d, ng):
    n = y2.shape[0]
    bn = 1000
    g = n // bn
    return pl.pallas_call(
        functools.partial(_final_body, ng=ng),
        grid=(g,),
        in_specs=[
            pl.BlockSpec((NC, bn, 128), lambda i: (0, i, 0)),
            pl.BlockSpec((bn, 128), lambda i: (i, 0)),
            pl.BlockSpec((bn, 1), lambda i: (i, 0)),
            pl.BlockSpec((128,), lambda i: (0,)),
            pl.BlockSpec((bn, 1), lambda i: (i, 0)),
        ],
        out_specs=pl.BlockSpec((ng, 128), lambda i: (0, 0)),
        out_shape=jax.ShapeDtypeStruct((ng, 128), jnp.float32),
        scratch_shapes=[
            pltpu.VMEM((ng, 128), jnp.float32),
            pltpu.VMEM((ng, 1), jnp.float32),
        ],
    )(z2p, y2, dinv, b2, batch2d)


def kernel(x, edge_index, batch, W1, b1, W2, b2):
    n = x.shape[0]
    ng = 64
    src = edge_index[0].astype(jnp.int32)
    dst = edge_index[1].astype(jnp.int32)
    batch2d = batch.astype(jnp.int32).reshape(n, 1)

    degp = _sc_degree(dst, n)
    degp = degp.reshape(NW, n).T
    dinv, y1 = _tc_prescale(degp, x)
    z1p = _sc_aggregate(y1, src, dst)
    y2 = _tc_mid(z1p, y1, dinv, W1, b1, W2)
    z2p = _sc_aggregate(y2, src, dst)
    return _tc_final(z2p, y2, dinv, b2, batch2d, ng)


# speedup vs baseline: 30.0808x; 1.8832x over previous
import dataclasses
import functools

import jax
import jax.numpy as jnp
from jax import lax
from jax.experimental import pallas as pl
from jax.experimental.pallas import tpu as pltpu
from jax.experimental.pallas import tpu_sc as plsc

def _sc_compiler_params():
    cp = pltpu.CompilerParams()
    if "needs_layout_passes" in pltpu.CompilerParams.__dataclass_fields__:
        cp = dataclasses.replace(cp, needs_layout_passes=False)
    return cp


NC = 2
NS = 16
NW = NC * NS
K = 80


def _sc_aggregate(y, src, dst):
    n, d = y.shape
    e = src.shape[0]
    e_per_w = e // NW
    chunks = e_per_w // K
    assert chunks % 2 == 1 and chunks >= 3
    pairs = (chunks - 1) // 2
    rps = (n // NS) // 8 * 8
    rlast = n - (NS - 1) * rps

    mesh = plsc.VectorSubcoreMesh(core_axis_name="c", subcore_axis_name="s")

    @functools.partial(
        pl.kernel,
        out_type=jax.ShapeDtypeStruct((NC, n, d), jnp.float32),
        mesh=mesh,
        scratch_types=[
            pltpu.VMEM((2, K), jnp.int32),
            pltpu.VMEM((2, K), jnp.int32),
            pltpu.VMEM((2, K, d), jnp.float32),
            pltpu.VMEM_SHARED((n, d), jnp.float32),
            pltpu.SemaphoreType.DMA,
            pltpu.SemaphoreType.DMA,
            pltpu.SemaphoreType.DMA,
            pltpu.SemaphoreType.DMA,
        ],
    )
    def k(y_hbm, src_hbm, dst_hbm, out_hbm, src_v, dst_v, rows_v, z_sh,
          si0, si1, sg0, sg1):
        cid = lax.axis_index("c")
        sid = lax.axis_index("s")
        wid = cid * NS + sid
        r0 = sid * rps
        base = wid * e_per_w
        s_i = (si0, si1)
        s_g = (sg0, sg1)

        def start_idx(c, b):
            off = base + c * K
            pltpu.async_copy(src_hbm.at[pl.ds(off, K)], src_v.at[b], s_i[b])
            pltpu.async_copy(dst_hbm.at[pl.ds(off, K)], dst_v.at[b], s_i[b])

        def wait_idx(c, b):
            off = base + c * K
            pltpu.make_async_copy(src_hbm.at[pl.ds(off, K)], src_v.at[b],
                                  s_i[b]).wait()
            pltpu.make_async_copy(dst_hbm.at[pl.ds(off, K)], dst_v.at[b],
                                  s_i[b]).wait()

        def start_gather(b):
            pltpu.async_copy(y_hbm.at[src_v.at[b]], rows_v.at[b], s_g[b])

        def wait_gather(b):
            pltpu.make_async_copy(y_hbm.at[src_v.at[b]], rows_v.at[b],
                                  s_g[b]).wait()

        def scatter(b):
            pltpu.sync_copy(rows_v.at[b], z_sh.at[dst_v.at[b]], add=True)

        start_idx(0, 0)
        start_idx(1, 1)

        @pl.when(sid < NS - 1)
        def _():
            pltpu.sync_copy(y_hbm.at[pl.ds(r0, rps)],
                            z_sh.at[pl.ds(r0, rps)])

        @pl.when(sid == NS - 1)
        def _():
            pltpu.sync_copy(y_hbm.at[pl.ds((NS - 1) * rps, rlast)],
                            z_sh.at[pl.ds((NS - 1) * rps, rlast)])

        wait_idx(0, 0)
        start_gather(0)
        plsc.subcore_barrier()

        @pl.loop(0, pairs)
        def _(j):
            for b in (0, 1):
                c = 2 * j + b
                nb = 1 - b
                wait_idx(c + 1, nb)
                start_gather(nb)
                wait_gather(b)
                scatter(b)
                if b == 0:
                    start_idx(c + 2, b)
                else:
                    @pl.when(j < pairs - 1)
                    def _():
                        start_idx(c + 2, b)

        wait_gather(0)
        scatter(0)

        plsc.subcore_barrier()

        @pl.when(sid < NS - 1)
        def _():
            pltpu.sync_copy(z_sh.at[pl.ds(r0, rps)],
                            out_hbm.at[cid, pl.ds(r0, rps)])

        @pl.when(sid == NS - 1)
        def _():
            pltpu.sync_copy(z_sh.at[pl.ds((NS - 1) * rps, rlast)],
                            out_hbm.at[cid, pl.ds((NS - 1) * rps, rlast)])

    return k(y, src, dst)


def _sc_degree(dst, n):
    e = dst.shape[0]
    e_per_w = e // NW

    mesh = plsc.VectorSubcoreMesh(core_axis_name="c", subcore_axis_name="s")

    @functools.partial(
        pl.kernel,
        out_type=jax.ShapeDtypeStruct((NW * n,), jnp.float32),
        mesh=mesh,
        scratch_types=[
            pltpu.VMEM((e_per_w,), jnp.int32),
            pltpu.VMEM((n,), jnp.float32),
        ],
        compiler_params=_sc_compiler_params(),
    )
    def k(dst_hbm, out_hbm, dst_v, deg_v):
        cid = lax.axis_index("c")
        sid = lax.axis_index("s")
        wid = cid * NS + sid
        pltpu.sync_copy(dst_hbm.at[pl.ds(wid * e_per_w, e_per_w)], dst_v)

        @pl.loop(0, n, step=16)
        def _(i):
            deg_v[pl.ds(i, 16)] = jnp.zeros((16,), jnp.float32)

        ones = jnp.ones((16,), jnp.float32)

        @pl.loop(0, e_per_w, step=16)
        def _(i):
            plsc.addupdate_scatter(deg_v, [dst_v[pl.ds(i, 16)]], ones)

        pltpu.sync_copy(deg_v, out_hbm.at[pl.ds(wid * n, n)])

    return k(dst)


def _prescale_body(degp_ref, x_ref, dinv_ref, y1_ref):
    deg = jnp.sum(degp_ref[...], axis=1, keepdims=True) + 1.0
    dinv = lax.rsqrt(deg)
    dinv_ref[...] = dinv
    y1_ref[...] = x_ref[...] * dinv


def _tc_prescale(degp, x):
    n = x.shape[0]
    bn = 1000
    g = n // bn
    return pl.pallas_call(
        _prescale_body,
        grid=(g,),
        in_specs=[
            pl.BlockSpec((bn, NW), lambda i: (i, 0)),
            pl.BlockSpec((bn, 128), lambda i: (i, 0)),
        ],
        out_specs=[
            pl.BlockSpec((bn, 1), lambda i: (i, 0)),
            pl.BlockSpec((bn, 128), lambda i: (i, 0)),
        ],
        out_shape=[
            jax.ShapeDtypeStruct((n, 1), jnp.float32),
            jax.ShapeDtypeStruct((n, 128), jnp.float32),
        ],
    )(degp, x)


def _mid_body(z1p_ref, y1_ref, dinv_ref, w1_ref, b1_ref, w2_ref, y2_ref):
    dinv = dinv_ref[...]
    a = (z1p_ref[0] + z1p_ref[1] - y1_ref[...]) * dinv
    pre = jnp.dot(a, w1_ref[...], preferred_element_type=jnp.float32)
    pre = pre + b1_ref[...][None, :]
    h = jnp.where(pre > 0, pre, 0.01 * pre)
    hw = jnp.dot(h, w2_ref[...], preferred_element_type=jnp.float32)
    y2_ref[...] = hw * dinv


def _tc_mid(z1p, y1, dinv, W1, b1, W2):
    n = y1.shape[0]
    bn = 1000
    g = n // bn
    dh = W1.shape[1]
    return pl.pallas_call(
        _mid_body,
        grid=(g,),
        in_specs=[
            pl.BlockSpec((NC, bn, 128), lambda i: (0, i, 0)),
            pl.BlockSpec((bn, 128), lambda i: (i, 0)),
            pl.BlockSpec((bn, 1), lambda i: (i, 0)),
            pl.BlockSpec((128, dh), lambda i: (0, 0)),
            pl.BlockSpec((dh,), lambda i: (0,)),
            pl.BlockSpec((dh, 128), lambda i: (0, 0)),
        ],
        out_specs=pl.BlockSpec((bn, 128), lambda i: (i, 0)),
        out_shape=jax.ShapeDtypeStruct((n, 128), jnp.float32),
    )(z1p, y1, dinv, W1, b1, W2)


def _final_body(z2p_ref, y2_ref, dinv_ref, b2_ref, batch_ref, out_ref,
                sums_ref, cnt_ref, ng):
    i = pl.program_id(0)
    bn = y2_ref.shape[0]
    ne = (z2p_ref[0] + z2p_ref[1] - y2_ref[...]) * dinv_ref[...]
    ne = ne + b2_ref[...][None, :]
    seg = batch_ref[...]
    oh = (seg == lax.broadcasted_iota(jnp.int32, (bn, ng), 1))
    oh = oh.astype(jnp.float32)
    dn = (((0,), (0,)), ((), ()))
    sums = lax.dot_general(oh, ne, dn, preferred_element_type=jnp.float32)
    cnt = lax.dot_general(oh, jnp.ones((bn, 1), jnp.float32), dn,
                          preferred_element_type=jnp.float32)

    @pl.when(i == 0)
    def _():
        sums_ref[...] = jnp.zeros_like(sums_ref)
        cnt_ref[...] = jnp.zeros_like(cnt_ref)

    sums_ref[...] += sums
    cnt_ref[...] += cnt

    @pl.when(i == pl.num_programs(0) - 1)
    def _():
        out_ref[...] = sums_ref[...] / jnp.maximum(cnt_ref[...], 1.0)


def _tc_final(z2p, y2, dinv, b2, batch2d, ng):
    n = y2.shape[0]
    bn = 1000
    g = n // bn
    return pl.pallas_call(
        functools.partial(_final_body, ng=ng),
        grid=(g,),
        in_specs=[
            pl.BlockSpec((NC, bn, 128), lambda i: (0, i, 0)),
            pl.BlockSpec((bn, 128), lambda i: (i, 0)),
            pl.BlockSpec((bn, 1), lambda i: (i, 0)),
            pl.BlockSpec((128,), lambda i: (0,)),
            pl.BlockSpec((bn, 1), lambda i: (i, 0)),
        ],
        out_specs=pl.BlockSpec((ng, 128), lambda i: (0, 0)),
        out_shape=jax.ShapeDtypeStruct((ng, 128), jnp.float32),
        scratch_shapes=[
            pltpu.VMEM((ng, 128), jnp.float32),
            pltpu.VMEM((ng, 1), jnp.float32),
        ],
    )(z2p, y2, dinv, b2, batch2d)


def kernel(x, edge_index, batch, W1, b1, W2, b2):
    n = x.shape[0]
    ng = 64
    src = edge_index[0].astype(jnp.int32)
    dst = edge_index[1].astype(jnp.int32)
    batch2d = batch.astype(jnp.int32).reshape(n, 1)

    degp = _sc_degree(dst, n)
    degp = degp.reshape(NW, n).T
    dinv, y1 = _tc_prescale(degp, x)
    z1p = _sc_aggregate(y1, src, dst)
    y2 = _tc_mid(z1p, y1, dinv, W1, b1, W2)
    z2p = _sc_aggregate(y2, src, dst)
    return _tc_final(z2p, y2, dinv, b2, batch2d, ng)


# speedup vs baseline: 35.2675x; 1.1724x over previous
import dataclasses
import functools

import jax
import jax.numpy as jnp
from jax import lax
from jax.experimental import pallas as pl
from jax.experimental.pallas import tpu as pltpu
from jax.experimental.pallas import tpu_sc as plsc

def _sc_compiler_params():
    cp = pltpu.CompilerParams()
    if "needs_layout_passes" in pltpu.CompilerParams.__dataclass_fields__:
        cp = dataclasses.replace(cp, needs_layout_passes=False)
    return cp


NC = 2
NS = 16
NW = NC * NS
K = 80


def _sc_aggregate(y, src, dst):
    n, d = y.shape
    e = src.shape[0]
    e_per_w = e // NW
    chunks = e_per_w // K
    assert chunks % 4 == 1 and chunks >= 5
    quads = (chunks - 1) // 4
    rps = (n // NS) // 8 * 8
    rlast = n - (NS - 1) * rps

    mesh = plsc.VectorSubcoreMesh(core_axis_name="c", subcore_axis_name="s")

    @functools.partial(
        pl.kernel,
        out_type=jax.ShapeDtypeStruct((NC, n, d), jnp.float32),
        mesh=mesh,
        scratch_types=[
            pltpu.VMEM((4, K), jnp.int32),
            pltpu.VMEM((4, K), jnp.int32),
            pltpu.VMEM((2, K, d), jnp.float32),
            pltpu.VMEM_SHARED((n, d), jnp.float32),
            pltpu.SemaphoreType.DMA,
            pltpu.SemaphoreType.DMA,
            pltpu.SemaphoreType.DMA,
            pltpu.SemaphoreType.DMA,
            pltpu.SemaphoreType.DMA,
            pltpu.SemaphoreType.DMA,
            pltpu.SemaphoreType.DMA,
            pltpu.SemaphoreType.DMA,
        ],
    )
    def k(y_hbm, src_hbm, dst_hbm, out_hbm, src_v, dst_v, rows_v, z_sh,
          si0, si1, si2, si3, sg0, sg1, ss0, ss1):
        cid = lax.axis_index("c")
        sid = lax.axis_index("s")
        wid = cid * NS + sid
        r0 = sid * rps
        base = wid * e_per_w
        s_i = (si0, si1, si2, si3)
        s_g = (sg0, sg1)
        s_s = (ss0, ss1)

        def start_idx(c, q):
            off = base + c * K
            pltpu.async_copy(src_hbm.at[pl.ds(off, K)], src_v.at[q], s_i[q])
            pltpu.async_copy(dst_hbm.at[pl.ds(off, K)], dst_v.at[q], s_i[q])

        def wait_idx(c, q):
            off = base + c * K
            pltpu.make_async_copy(src_hbm.at[pl.ds(off, K)], src_v.at[q],
                                  s_i[q]).wait()
            pltpu.make_async_copy(dst_hbm.at[pl.ds(off, K)], dst_v.at[q],
                                  s_i[q]).wait()

        def start_gather(q, b):
            pltpu.async_copy(y_hbm.at[src_v.at[q]], rows_v.at[b], s_g[b])

        def wait_gather(q, b):
            pltpu.make_async_copy(y_hbm.at[src_v.at[q]], rows_v.at[b],
                                  s_g[b]).wait()

        def start_scatter(q, b):
            pltpu.async_copy(rows_v.at[b], z_sh.at[dst_v.at[q]], s_s[b],
                             add=True)

        def wait_scatter(q, b):
            pltpu.make_async_copy(rows_v.at[b], z_sh.at[dst_v.at[q]],
                                  s_s[b]).wait()

        for q in range(4):
            start_idx(q, q)

        @pl.when(sid < NS - 1)
        def _():
            pltpu.sync_copy(y_hbm.at[pl.ds(r0, rps)],
                            z_sh.at[pl.ds(r0, rps)])

        @pl.when(sid == NS - 1)
        def _():
            pltpu.sync_copy(y_hbm.at[pl.ds((NS - 1) * rps, rlast)],
                            z_sh.at[pl.ds((NS - 1) * rps, rlast)])

        wait_idx(0, 0)
        start_gather(0, 0)
        plsc.subcore_barrier()

        @pl.loop(0, quads)
        def _(j):
            for q in range(4):
                c = 4 * j + q
                b = q % 2
                nb = 1 - b
                nq = (q + 1) % 4
                pq = (q + 3) % 4
                if q == 0:
                    @pl.when(j > 0)
                    def _():
                        wait_scatter(pq, nb)
                        start_idx(c + 3, pq)
                else:
                    wait_scatter(pq, nb)
                    if q <= 1:
                        start_idx(c + 3, pq)
                    else:
                        @pl.when(j < quads - 1)
                        def _():
                            start_idx(c + 3, pq)
                wait_idx(c + 1, nq)
                start_gather(nq, nb)
                wait_gather(q, b)
                start_scatter(q, b)

        wait_scatter(3, 1)
        wait_gather(0, 0)
        pltpu.sync_copy(rows_v.at[0], z_sh.at[dst_v.at[0]], add=True)

        plsc.subcore_barrier()

        @pl.when(sid < NS - 1)
        def _():
            pltpu.sync_copy(z_sh.at[pl.ds(r0, rps)],
                            out_hbm.at[cid, pl.ds(r0, rps)])

        @pl.when(sid == NS - 1)
        def _():
            pltpu.sync_copy(z_sh.at[pl.ds((NS - 1) * rps, rlast)],
                            out_hbm.at[cid, pl.ds((NS - 1) * rps, rlast)])

    return k(y, src, dst)


def _sc_degree(dst, n):
    e = dst.shape[0]
    e_per_w = e // NW

    mesh = plsc.VectorSubcoreMesh(core_axis_name="c", subcore_axis_name="s")

    @functools.partial(
        pl.kernel,
        out_type=jax.ShapeDtypeStruct((NW * n,), jnp.float32),
        mesh=mesh,
        scratch_types=[
            pltpu.VMEM((e_per_w,), jnp.int32),
            pltpu.VMEM((n,), jnp.float32),
        ],
        compiler_params=_sc_compiler_params(),
    )
    def k(dst_hbm, out_hbm, dst_v, deg_v):
        cid = lax.axis_index("c")
        sid = lax.axis_index("s")
        wid = cid * NS + sid
        pltpu.sync_copy(dst_hbm.at[pl.ds(wid * e_per_w, e_per_w)], dst_v)

        @pl.loop(0, n, step=16)
        def _(i):
            deg_v[pl.ds(i, 16)] = jnp.zeros((16,), jnp.float32)

        ones = jnp.ones((16,), jnp.float32)

        @pl.loop(0, e_per_w, step=16)
        def _(i):
            plsc.addupdate_scatter(deg_v, [dst_v[pl.ds(i, 16)]], ones)

        pltpu.sync_copy(deg_v, out_hbm.at[pl.ds(wid * n, n)])

    return k(dst)


def _prescale_body(degp_ref, x_ref, dinv_ref, y1_ref):
    deg = jnp.sum(degp_ref[...], axis=1, keepdims=True) + 1.0
    dinv = lax.rsqrt(deg)
    dinv_ref[...] = dinv
    y1_ref[...] = x_ref[...] * dinv


def _tc_prescale(degp, x):
    n = x.shape[0]
    bn = 1000
    g = n // bn
    return pl.pallas_call(
        _prescale_body,
        grid=(g,),
        in_specs=[
            pl.BlockSpec((bn, NW), lambda i: (i, 0)),
            pl.BlockSpec((bn, 128), lambda i: (i, 0)),
        ],
        out_specs=[
            pl.BlockSpec((bn, 1), lambda i: (i, 0)),
            pl.BlockSpec((bn, 128), lambda i: (i, 0)),
        ],
        out_shape=[
            jax.ShapeDtypeStruct((n, 1), jnp.float32),
            jax.ShapeDtypeStruct((n, 128), jnp.float32),
        ],
    )(degp, x)


def _mid_body(z1p_ref, y1_ref, dinv_ref, w1_ref, b1_ref, w2_ref, y2_ref):
    dinv = dinv_ref[...]
    a = (z1p_ref[0] + z1p_ref[1] - y1_ref[...]) * dinv
    pre = jnp.dot(a, w1_ref[...], preferred_element_type=jnp.float32)
    pre = pre + b1_ref[...][None, :]
    h = jnp.where(pre > 0, pre, 0.01 * pre)
    hw = jnp.dot(h, w2_ref[...], preferred_element_type=jnp.float32)
    y2_ref[...] = hw * dinv


def _tc_mid(z1p, y1, dinv, W1, b1, W2):
    n = y1.shape[0]
    bn = 1000
    g = n // bn
    dh = W1.shape[1]
    return pl.pallas_call(
        _mid_body,
        grid=(g,),
        in_specs=[
            pl.BlockSpec((NC, bn, 128), lambda i: (0, i, 0)),
            pl.BlockSpec((bn, 128), lambda i: (i, 0)),
            pl.BlockSpec((bn, 1), lambda i: (i, 0)),
            pl.BlockSpec((128, dh), lambda i: (0, 0)),
            pl.BlockSpec((dh,), lambda i: (0,)),
            pl.BlockSpec((dh, 128), lambda i: (0, 0)),
        ],
        out_specs=pl.BlockSpec((bn, 128), lambda i: (i, 0)),
        out_shape=jax.ShapeDtypeStruct((n, 128), jnp.float32),
    )(z1p, y1, dinv, W1, b1, W2)


def _final_body(z2p_ref, y2_ref, dinv_ref, b2_ref, batch_ref, out_ref,
                sums_ref, cnt_ref, ng):
    i = pl.program_id(0)
    bn = y2_ref.shape[0]
    ne = (z2p_ref[0] + z2p_ref[1] - y2_ref[...]) * dinv_ref[...]
    ne = ne + b2_ref[...][None, :]
    seg = batch_ref[...]
    oh = (seg == lax.broadcasted_iota(jnp.int32, (bn, ng), 1))
    oh = oh.astype(jnp.float32)
    dn = (((0,), (0,)), ((), ()))
    sums = lax.dot_general(oh, ne, dn, preferred_element_type=jnp.float32)
    cnt = lax.dot_general(oh, jnp.ones((bn, 1), jnp.float32), dn,
                          preferred_element_type=jnp.float32)

    @pl.when(i == 0)
    def _():
        sums_ref[...] = jnp.zeros_like(sums_ref)
        cnt_ref[...] = jnp.zeros_like(cnt_ref)

    sums_ref[...] += sums
    cnt_ref[...] += cnt

    @pl.when(i == pl.num_programs(0) - 1)
    def _():
        out_ref[...] = sums_ref[...] / jnp.maximum(cnt_ref[...], 1.0)


def _tc_final(z2p, y2, dinv, b2, batch2d, ng):
    n = y2.shape[0]
    bn = 1000
    g = n // bn
    return pl.pallas_call(
        functools.partial(_final_body, ng=ng),
        grid=(g,),
        in_specs=[
            pl.BlockSpec((NC, bn, 128), lambda i: (0, i, 0)),
            pl.BlockSpec((bn, 128), lambda i: (i, 0)),
            pl.BlockSpec((bn, 1), lambda i: (i, 0)),
            pl.BlockSpec((128,), lambda i: (0,)),
            pl.BlockSpec((bn, 1), lambda i: (i, 0)),
        ],
        out_specs=pl.BlockSpec((ng, 128), lambda i: (0, 0)),
        out_shape=jax.ShapeDtypeStruct((ng, 128), jnp.float32),
        scratch_shapes=[
            pltpu.VMEM((ng, 128), jnp.float32),
            pltpu.VMEM((ng, 1), jnp.float32),
        ],
    )(z2p, y2, dinv, b2, batch2d)


def kernel(x, edge_index, batch, W1, b1, W2, b2):
    n = x.shape[0]
    ng = 64
    src = edge_index[0].astype(jnp.int32)
    dst = edge_index[1].astype(jnp.int32)
    batch2d = batch.astype(jnp.int32).reshape(n, 1)

    degp = _sc_degree(dst, n)
    degp = degp.reshape(NW, n).T
    dinv, y1 = _tc_prescale(degp, x)
    z1p = _sc_aggregate(y1, src, dst)
    y2 = _tc_mid(z1p, y1, dinv, W1, b1, W2)
    z2p = _sc_aggregate(y2, src, dst)
    return _tc_final(z2p, y2, dinv, b2, batch2d, ng)


# speedup vs baseline: 37.5996x; 1.0661x over previous
import dataclasses
import functools

import jax
import jax.numpy as jnp
from jax import lax
from jax.experimental import pallas as pl
from jax.experimental.pallas import tpu as pltpu
from jax.experimental.pallas import tpu_sc as plsc

def _sc_compiler_params():
    cp = pltpu.CompilerParams()
    if "needs_layout_passes" in pltpu.CompilerParams.__dataclass_fields__:
        cp = dataclasses.replace(cp, needs_layout_passes=False)
    return cp


NC = 2
NS = 16
NW = NC * NS
K = 80


def _sc_aggregate(y, ei_flat):
    n, d = y.shape
    e = ei_flat.shape[0] // 2
    e_per_w = e // NW
    chunks = e_per_w // K
    assert chunks % 4 == 1 and chunks >= 5
    quads = (chunks - 1) // 4
    rps = (n // NS) // 8 * 8
    rlast = n - (NS - 1) * rps

    mesh = plsc.VectorSubcoreMesh(core_axis_name="c", subcore_axis_name="s")

    @functools.partial(
        pl.kernel,
        out_type=jax.ShapeDtypeStruct((NC, n, d), jnp.float32),
        mesh=mesh,
        scratch_types=[
            pltpu.VMEM((4, K), jnp.int32),
            pltpu.VMEM((4, K), jnp.int32),
            pltpu.VMEM((2, K, d), jnp.float32),
            pltpu.VMEM_SHARED((n, d), jnp.float32),
            pltpu.SemaphoreType.DMA,
            pltpu.SemaphoreType.DMA,
            pltpu.SemaphoreType.DMA,
            pltpu.SemaphoreType.DMA,
            pltpu.SemaphoreType.DMA,
            pltpu.SemaphoreType.DMA,
            pltpu.SemaphoreType.DMA,
            pltpu.SemaphoreType.DMA,
        ],
    )
    def k(y_hbm, ei_hbm, out_hbm, src_v, dst_v, rows_v, z_sh,
          si0, si1, si2, si3, sg0, sg1, ss0, ss1):
        cid = lax.axis_index("c")
        sid = lax.axis_index("s")
        wid = cid * NS + sid
        r0 = sid * rps
        base = wid * e_per_w
        s_i = (si0, si1, si2, si3)
        s_g = (sg0, sg1)
        s_s = (ss0, ss1)

        def start_idx(c, q):
            off = base + c * K
            pltpu.async_copy(ei_hbm.at[pl.ds(off, K)], src_v.at[q], s_i[q])
            pltpu.async_copy(ei_hbm.at[pl.ds(e + off, K)], dst_v.at[q],
                             s_i[q])

        def wait_idx(c, q):
            off = base + c * K
            pltpu.make_async_copy(ei_hbm.at[pl.ds(off, K)], src_v.at[q],
                                  s_i[q]).wait()
            pltpu.make_async_copy(ei_hbm.at[pl.ds(e + off, K)], dst_v.at[q],
                                  s_i[q]).wait()

        def start_gather(q, b):
            pltpu.async_copy(y_hbm.at[src_v.at[q]], rows_v.at[b], s_g[b])

        def wait_gather(q, b):
            pltpu.make_async_copy(y_hbm.at[src_v.at[q]], rows_v.at[b],
                                  s_g[b]).wait()

        def start_scatter(q, b):
            pltpu.async_copy(rows_v.at[b], z_sh.at[dst_v.at[q]], s_s[b],
                             add=True)

        def wait_scatter(q, b):
            pltpu.make_async_copy(rows_v.at[b], z_sh.at[dst_v.at[q]],
                                  s_s[b]).wait()

        for q in range(4):
            start_idx(q, q)

        @pl.when(sid < NS - 1)
        def _():
            pltpu.sync_copy(y_hbm.at[pl.ds(r0, rps)],
                            z_sh.at[pl.ds(r0, rps)])

        @pl.when(sid == NS - 1)
        def _():
            pltpu.sync_copy(y_hbm.at[pl.ds((NS - 1) * rps, rlast)],
                            z_sh.at[pl.ds((NS - 1) * rps, rlast)])

        wait_idx(0, 0)
        start_gather(0, 0)
        plsc.subcore_barrier()

        @pl.loop(0, quads)
        def _(j):
            for q in range(4):
                c = 4 * j + q
                b = q % 2
                nb = 1 - b
                nq = (q + 1) % 4
                pq = (q + 3) % 4
                if q == 0:
                    @pl.when(j > 0)
                    def _():
                        wait_scatter(pq, nb)
                        start_idx(c + 3, pq)
                else:
                    wait_scatter(pq, nb)
                    if q <= 1:
                        start_idx(c + 3, pq)
                    else:
                        @pl.when(j < quads - 1)
                        def _():
                            start_idx(c + 3, pq)
                wait_idx(c + 1, nq)
                start_gather(nq, nb)
                wait_gather(q, b)
                start_scatter(q, b)

        wait_scatter(3, 1)
        wait_gather(0, 0)
        pltpu.sync_copy(rows_v.at[0], z_sh.at[dst_v.at[0]], add=True)

        plsc.subcore_barrier()

        @pl.when(sid < NS - 1)
        def _():
            pltpu.sync_copy(z_sh.at[pl.ds(r0, rps)],
                            out_hbm.at[cid, pl.ds(r0, rps)])

        @pl.when(sid == NS - 1)
        def _():
            pltpu.sync_copy(z_sh.at[pl.ds((NS - 1) * rps, rlast)],
                            out_hbm.at[cid, pl.ds((NS - 1) * rps, rlast)])

    return k(y, ei_flat)


def _sc_degree(ei_flat, n):
    e = ei_flat.shape[0] // 2
    e_per_w = e // NW

    mesh = plsc.VectorSubcoreMesh(core_axis_name="c", subcore_axis_name="s")

    @functools.partial(
        pl.kernel,
        out_type=jax.ShapeDtypeStruct((NW * n,), jnp.float32),
        mesh=mesh,
        scratch_types=[
            pltpu.VMEM((e_per_w,), jnp.int32),
            pltpu.VMEM((n,), jnp.float32),
        ],
        compiler_params=_sc_compiler_params(),
    )
    def k(ei_hbm, out_hbm, dst_v, deg_v):
        cid = lax.axis_index("c")
        sid = lax.axis_index("s")
        wid = cid * NS + sid
        pltpu.sync_copy(ei_hbm.at[pl.ds(e + wid * e_per_w, e_per_w)], dst_v)

        @pl.loop(0, n, step=16)
        def _(i):
            deg_v[pl.ds(i, 16)] = jnp.zeros((16,), jnp.float32)

        ones = jnp.ones((16,), jnp.float32)

        @pl.loop(0, e_per_w, step=16)
        def _(i):
            plsc.addupdate_scatter(deg_v, [dst_v[pl.ds(i, 16)]], ones)

        pltpu.sync_copy(deg_v, out_hbm.at[pl.ds(wid * n, n)])

    return k(ei_flat)


def _prescale_body(degp_ref, x_ref, dinv_ref, y1_ref):
    deg = jnp.sum(degp_ref[...], axis=1, keepdims=True) + 1.0
    dinv = lax.rsqrt(deg)
    dinv_ref[...] = dinv
    y1_ref[...] = x_ref[...] * dinv


def _tc_prescale(degp, x):
    n = x.shape[0]
    bn = 2000
    g = n // bn
    return pl.pallas_call(
        _prescale_body,
        grid=(g,),
        in_specs=[
            pl.BlockSpec((bn, NW), lambda i: (i, 0)),
            pl.BlockSpec((bn, 128), lambda i: (i, 0)),
        ],
        out_specs=[
            pl.BlockSpec((bn, 1), lambda i: (i, 0)),
            pl.BlockSpec((bn, 128), lambda i: (i, 0)),
        ],
        out_shape=[
            jax.ShapeDtypeStruct((n, 1), jnp.float32),
            jax.ShapeDtypeStruct((n, 128), jnp.float32),
        ],
    )(degp, x)


def _mid_body(z1p_ref, y1_ref, dinv_ref, w1_ref, b1_ref, w2_ref, y2_ref):
    dinv = dinv_ref[...]
    a = (z1p_ref[0] + z1p_ref[1] - y1_ref[...]) * dinv
    pre = jnp.dot(a.astype(jnp.bfloat16), w1_ref[...].astype(jnp.bfloat16),
                  preferred_element_type=jnp.float32)
    pre = pre + b1_ref[...][None, :]
    h = jnp.where(pre > 0, pre, 0.01 * pre)
    hw = jnp.dot(h.astype(jnp.bfloat16), w2_ref[...].astype(jnp.bfloat16),
                 preferred_element_type=jnp.float32)
    y2_ref[...] = hw * dinv


def _tc_mid(z1p, y1, dinv, W1, b1, W2):
    n = y1.shape[0]
    bn = 2000
    g = n // bn
    dh = W1.shape[1]
    return pl.pallas_call(
        _mid_body,
        grid=(g,),
        in_specs=[
            pl.BlockSpec((NC, bn, 128), lambda i: (0, i, 0)),
            pl.BlockSpec((bn, 128), lambda i: (i, 0)),
            pl.BlockSpec((bn, 1), lambda i: (i, 0)),
            pl.BlockSpec((128, dh), lambda i: (0, 0)),
            pl.BlockSpec((dh,), lambda i: (0,)),
            pl.BlockSpec((dh, 128), lambda i: (0, 0)),
        ],
        out_specs=pl.BlockSpec((bn, 128), lambda i: (i, 0)),
        out_shape=jax.ShapeDtypeStruct((n, 128), jnp.float32),
    )(z1p, y1, dinv, W1, b1, W2)


def _final_body(z2p_ref, y2_ref, dinv_ref, b2_ref, batch_ref, out_ref,
                sums_ref, cnt_ref, ng):
    i = pl.program_id(0)
    bn = y2_ref.shape[0]
    ne = (z2p_ref[0] + z2p_ref[1] - y2_ref[...]) * dinv_ref[...]
    ne = ne + b2_ref[...][None, :]
    seg = batch_ref[0]
    oh = (seg == lax.broadcasted_iota(jnp.int32, (ng, bn), 0))
    oh = oh.astype(jnp.float32)
    dn = (((1,), (0,)), ((), ()))
    sums = lax.dot_general(oh, ne, dn, preferred_element_type=jnp.float32)
    cnt = lax.dot_general(oh, jnp.ones((bn, 1), jnp.float32), dn,
                          preferred_element_type=jnp.float32)

    @pl.when(i == 0)
    def _():
        sums_ref[...] = jnp.zeros_like(sums_ref)
        cnt_ref[...] = jnp.zeros_like(cnt_ref)

    sums_ref[...] += sums
    cnt_ref[...] += cnt

    @pl.when(i == pl.num_programs(0) - 1)
    def _():
        out_ref[...] = sums_ref[...] / jnp.maximum(cnt_ref[...], 1.0)


def _tc_final(z2p, y2, dinv, b2, batch_rows, ng):
    n = y2.shape[0]
    bn = 2000
    g = n // bn
    return pl.pallas_call(
        functools.partial(_final_body, ng=ng),
        grid=(g,),
        in_specs=[
            pl.BlockSpec((NC, bn, 128), lambda i: (0, i, 0)),
            pl.BlockSpec((bn, 128), lambda i: (i, 0)),
            pl.BlockSpec((bn, 1), lambda i: (i, 0)),
            pl.BlockSpec((128,), lambda i: (0,)),
            pl.BlockSpec((1, 1, bn), lambda i: (i, 0, 0)),
        ],
        out_specs=pl.BlockSpec((ng, 128), lambda i: (0, 0)),
        out_shape=jax.ShapeDtypeStruct((ng, 128), jnp.float32),
        scratch_shapes=[
            pltpu.VMEM((ng, 128), jnp.float32),
            pltpu.VMEM((ng, 1), jnp.float32),
        ],
    )(z2p, y2, dinv, b2, batch_rows)


def kernel(x, edge_index, batch, W1, b1, W2, b2):
    n = x.shape[0]
    ng = 64
    ei_flat = edge_index.astype(jnp.int32).reshape(-1)
    batch_rows = batch.astype(jnp.int32).reshape(n // 2000, 1, 2000)

    degp = _sc_degree(ei_flat, n)
    degp = degp.reshape(NW, n).T
    dinv, y1 = _tc_prescale(degp, x)
    z1p = _sc_aggregate(y1, ei_flat)
    y2 = _tc_mid(z1p, y1, dinv, W1, b1, W2)
    z2p = _sc_aggregate(y2, ei_flat)
    return _tc_final(z2p, y2, dinv, b2, batch_rows, ng)


# speedup vs baseline: 40.2164x; 1.0696x over previous
import dataclasses
import functools

import jax
import jax.numpy as jnp
from jax import lax
from jax.experimental import pallas as pl
from jax.experimental.pallas import tpu as pltpu
from jax.experimental.pallas import tpu_sc as plsc

def _sc_compiler_params():
    cp = pltpu.CompilerParams()
    if "needs_layout_passes" in pltpu.CompilerParams.__dataclass_fields__:
        cp = dataclasses.replace(cp, needs_layout_passes=False)
    return cp


NC = 2
NS = 16
NW = NC * NS
K = 112


def _sc_aggregate(y, ei_flat):
    n, d = y.shape
    e = ei_flat.shape[0] // 2
    e_per_w = e // NW
    chunks = e_per_w // K
    kt = e_per_w - chunks * K
    assert chunks % 4 == 1 and chunks >= 5
    assert 0 < kt <= 128 and kt % 8 == 0
    quads = (chunks - 1) // 4
    rps = (n // NS) // 8 * 8
    rlast = n - (NS - 1) * rps

    mesh = plsc.VectorSubcoreMesh(core_axis_name="c", subcore_axis_name="s")

    @functools.partial(
        pl.kernel,
        out_type=jax.ShapeDtypeStruct((NC, n, d), jnp.float32),
        mesh=mesh,
        scratch_types=[
            pltpu.VMEM((4, K), jnp.int32),
            pltpu.VMEM((4, K), jnp.int32),
            pltpu.VMEM((2, K, d), jnp.float32),
            pltpu.VMEM((kt,), jnp.int32),
            pltpu.VMEM((kt,), jnp.int32),
            pltpu.VMEM((kt, d), jnp.float32),
            pltpu.VMEM_SHARED((n, d), jnp.float32),
            pltpu.SemaphoreType.DMA,
            pltpu.SemaphoreType.DMA,
            pltpu.SemaphoreType.DMA,
            pltpu.SemaphoreType.DMA,
            pltpu.SemaphoreType.DMA,
            pltpu.SemaphoreType.DMA,
            pltpu.SemaphoreType.DMA,
            pltpu.SemaphoreType.DMA,
            pltpu.SemaphoreType.DMA,
            pltpu.SemaphoreType.DMA,
        ],
    )
    def k(y_hbm, ei_hbm, out_hbm, src_v, dst_v, rows_v, src_t, dst_t, rows_t,
          z_sh, si0, si1, si2, si3, sg0, sg1, ss0, ss1, sit, sgt):
        cid = lax.axis_index("c")
        sid = lax.axis_index("s")
        wid = cid * NS + sid
        r0 = sid * rps
        base = wid * e_per_w
        s_i = (si0, si1, si2, si3)
        s_g = (sg0, sg1)
        s_s = (ss0, ss1)

        def start_idx(c, q):
            off = base + c * K
            pltpu.async_copy(ei_hbm.at[pl.ds(off, K)], src_v.at[q], s_i[q])
            pltpu.async_copy(ei_hbm.at[pl.ds(e + off, K)], dst_v.at[q],
                             s_i[q])

        def wait_idx(c, q):
            off = base + c * K
            pltpu.make_async_copy(ei_hbm.at[pl.ds(off, K)], src_v.at[q],
                                  s_i[q]).wait()
            pltpu.make_async_copy(ei_hbm.at[pl.ds(e + off, K)], dst_v.at[q],
                                  s_i[q]).wait()

        def start_gather(q, b):
            pltpu.async_copy(y_hbm.at[src_v.at[q]], rows_v.at[b], s_g[b])

        def wait_gather(q, b):
            pltpu.make_async_copy(y_hbm.at[src_v.at[q]], rows_v.at[b],
                                  s_g[b]).wait()

        def start_scatter(q, b):
            pltpu.async_copy(rows_v.at[b], z_sh.at[dst_v.at[q]], s_s[b],
                             add=True)

        def wait_scatter(q, b):
            pltpu.make_async_copy(rows_v.at[b], z_sh.at[dst_v.at[q]],
                                  s_s[b]).wait()

        for q in range(4):
            start_idx(q, q)
        toff = base + chunks * K
        pltpu.async_copy(ei_hbm.at[pl.ds(toff, kt)], src_t, sit)
        pltpu.async_copy(ei_hbm.at[pl.ds(e + toff, kt)], dst_t, sit)

        @pl.when(sid < NS - 1)
        def _():
            pltpu.sync_copy(y_hbm.at[pl.ds(r0, rps)],
                            z_sh.at[pl.ds(r0, rps)])

        @pl.when(sid == NS - 1)
        def _():
            pltpu.sync_copy(y_hbm.at[pl.ds((NS - 1) * rps, rlast)],
                            z_sh.at[pl.ds((NS - 1) * rps, rlast)])

        wait_idx(0, 0)
        start_gather(0, 0)
        pltpu.make_async_copy(ei_hbm.at[pl.ds(toff, kt)], src_t, sit).wait()
        pltpu.make_async_copy(ei_hbm.at[pl.ds(e + toff, kt)], dst_t,
                              sit).wait()
        pltpu.async_copy(y_hbm.at[src_t], rows_t, sgt)
        plsc.subcore_barrier()

        @pl.loop(0, quads)
        def _(j):
            for q in range(4):
                c = 4 * j + q
                b = q % 2
                nb = 1 - b
                nq = (q + 1) % 4
                pq = (q + 3) % 4
                if q == 0:
                    @pl.when(j > 0)
                    def _():
                        wait_scatter(pq, nb)
                        start_idx(c + 3, pq)
                else:
                    wait_scatter(pq, nb)
                    if q <= 1:
                        start_idx(c + 3, pq)
                    else:
                        @pl.when(j < quads - 1)
                        def _():
                            start_idx(c + 3, pq)
                wait_idx(c + 1, nq)
                start_gather(nq, nb)
                wait_gather(q, b)
                start_scatter(q, b)

        wait_scatter(3, 1)
        wait_gather(0, 0)
        pltpu.sync_copy(rows_v.at[0], z_sh.at[dst_v.at[0]], add=True)
        pltpu.make_async_copy(y_hbm.at[src_t], rows_t, sgt).wait()
        pltpu.sync_copy(rows_t, z_sh.at[dst_t], add=True)

        plsc.subcore_barrier()

        @pl.when(sid < NS - 1)
        def _():
            pltpu.sync_copy(z_sh.at[pl.ds(r0, rps)],
                            out_hbm.at[cid, pl.ds(r0, rps)])

        @pl.when(sid == NS - 1)
        def _():
            pltpu.sync_copy(z_sh.at[pl.ds((NS - 1) * rps, rlast)],
                            out_hbm.at[cid, pl.ds((NS - 1) * rps, rlast)])

    return k(y, ei_flat)


def _sc_degree(ei_flat, n):
    e = ei_flat.shape[0] // 2
    e_per_w = e // NW

    mesh = plsc.VectorSubcoreMesh(core_axis_name="c", subcore_axis_name="s")

    @functools.partial(
        pl.kernel,
        out_type=jax.ShapeDtypeStruct((NW * n,), jnp.float32),
        mesh=mesh,
        scratch_types=[
            pltpu.VMEM((e_per_w,), jnp.int32),
            pltpu.VMEM((n,), jnp.float32),
        ],
        compiler_params=_sc_compiler_params(),
    )
    def k(ei_hbm, out_hbm, dst_v, deg_v):
        cid = lax.axis_index("c")
        sid = lax.axis_index("s")
        wid = cid * NS + sid
        pltpu.sync_copy(ei_hbm.at[pl.ds(e + wid * e_per_w, e_per_w)], dst_v)

        @pl.loop(0, n, step=16)
        def _(i):
            deg_v[pl.ds(i, 16)] = jnp.zeros((16,), jnp.float32)

        ones = jnp.ones((16,), jnp.float32)

        @pl.loop(0, e_per_w, step=16)
        def _(i):
            plsc.addupdate_scatter(deg_v, [dst_v[pl.ds(i, 16)]], ones)

        pltpu.sync_copy(deg_v, out_hbm.at[pl.ds(wid * n, n)])

    return k(ei_flat)


def _prescale_body(degp_ref, x_ref, dinv_ref, y1_ref):
    deg = jnp.sum(degp_ref[...], axis=1, keepdims=True) + 1.0
    dinv = lax.rsqrt(deg)
    dinv_ref[...] = dinv
    y1_ref[...] = x_ref[...] * dinv


def _tc_prescale(degp, x):
    n = x.shape[0]
    bn = 2000
    g = n // bn
    return pl.pallas_call(
        _prescale_body,
        grid=(g,),
        in_specs=[
            pl.BlockSpec((bn, NW), lambda i: (i, 0)),
            pl.BlockSpec((bn, 128), lambda i: (i, 0)),
        ],
        out_specs=[
            pl.BlockSpec((bn, 1), lambda i: (i, 0)),
            pl.BlockSpec((bn, 128), lambda i: (i, 0)),
        ],
        out_shape=[
            jax.ShapeDtypeStruct((n, 1), jnp.float32),
            jax.ShapeDtypeStruct((n, 128), jnp.float32),
        ],
    )(degp, x)


def _mid_body(z1p_ref, y1_ref, dinv_ref, w1_ref, b1_ref, w2_ref, y2_ref):
    dinv = dinv_ref[...]
    a = (z1p_ref[0] + z1p_ref[1] - y1_ref[...]) * dinv
    pre = jnp.dot(a.astype(jnp.bfloat16), w1_ref[...].astype(jnp.bfloat16),
                  preferred_element_type=jnp.float32)
    pre = pre + b1_ref[...][None, :]
    h = jnp.where(pre > 0, pre, 0.01 * pre)
    hw = jnp.dot(h.astype(jnp.bfloat16), w2_ref[...].astype(jnp.bfloat16),
                 preferred_element_type=jnp.float32)
    y2_ref[...] = hw * dinv


def _tc_mid(z1p, y1, dinv, W1, b1, W2):
    n = y1.shape[0]
    bn = 2000
    g = n // bn
    dh = W1.shape[1]
    return pl.pallas_call(
        _mid_body,
        grid=(g,),
        in_specs=[
            pl.BlockSpec((NC, bn, 128), lambda i: (0, i, 0)),
            pl.BlockSpec((bn, 128), lambda i: (i, 0)),
            pl.BlockSpec((bn, 1), lambda i: (i, 0)),
            pl.BlockSpec((128, dh), lambda i: (0, 0)),
            pl.BlockSpec((dh,), lambda i: (0,)),
            pl.BlockSpec((dh, 128), lambda i: (0, 0)),
        ],
        out_specs=pl.BlockSpec((bn, 128), lambda i: (i, 0)),
        out_shape=jax.ShapeDtypeStruct((n, 128), jnp.float32),
    )(z1p, y1, dinv, W1, b1, W2)


def _final_body(z2p_ref, y2_ref, dinv_ref, b2_ref, batch_ref, out_ref,
                sums_ref, cnt_ref, ng):
    i = pl.program_id(0)
    bn = y2_ref.shape[0]
    ne = (z2p_ref[0] + z2p_ref[1] - y2_ref[...]) * dinv_ref[...]
    ne = ne + b2_ref[...][None, :]
    seg = batch_ref[0]
    oh = (seg == lax.broadcasted_iota(jnp.int32, (ng, bn), 0))
    oh = oh.astype(jnp.float32)
    dn = (((1,), (0,)), ((), ()))
    sums = lax.dot_general(oh, ne, dn, preferred_element_type=jnp.float32)
    cnt = lax.dot_general(oh, jnp.ones((bn, 1), jnp.float32), dn,
                          preferred_element_type=jnp.float32)

    @pl.when(i == 0)
    def _():
        sums_ref[...] = jnp.zeros_like(sums_ref)
        cnt_ref[...] = jnp.zeros_like(cnt_ref)

    sums_ref[...] += sums
    cnt_ref[...] += cnt

    @pl.when(i == pl.num_programs(0) - 1)
    def _():
        out_ref[...] = sums_ref[...] / jnp.maximum(cnt_ref[...], 1.0)


def _tc_final(z2p, y2, dinv, b2, batch_rows, ng):
    n = y2.shape[0]
    bn = 2000
    g = n // bn
    return pl.pallas_call(
        functools.partial(_final_body, ng=ng),
        grid=(g,),
        in_specs=[
            pl.BlockSpec((NC, bn, 128), lambda i: (0, i, 0)),
            pl.BlockSpec((bn, 128), lambda i: (i, 0)),
            pl.BlockSpec((bn, 1), lambda i: (i, 0)),
            pl.BlockSpec((128,), lambda i: (0,)),
            pl.BlockSpec((1, 1, bn), lambda i: (i, 0, 0)),
        ],
        out_specs=pl.BlockSpec((ng, 128), lambda i: (0, 0)),
        out_shape=jax.ShapeDtypeStruct((ng, 128), jnp.float32),
        scratch_shapes=[
            pltpu.VMEM((ng, 128), jnp.float32),
            pltpu.VMEM((ng, 1), jnp.float32),
        ],
    )(z2p, y2, dinv, b2, batch_rows)


def kernel(x, edge_index, batch, W1, b1, W2, b2):
    n = x.shape[0]
    ng = 64
    ei_flat = edge_index.astype(jnp.int32).reshape(-1)
    batch_rows = batch.astype(jnp.int32).reshape(n // 2000, 1, 2000)

    degp = _sc_degree(ei_flat, n)
    degp = degp.reshape(NW, n).T
    dinv, y1 = _tc_prescale(degp, x)
    z1p = _sc_aggregate(y1, ei_flat)
    y2 = _tc_mid(z1p, y1, dinv, W1, b1, W2)
    z2p = _sc_aggregate(y2, ei_flat)
    return _tc_final(z2p, y2, dinv, b2, batch_rows, ng)


# speedup vs baseline: 40.4940x; 1.0069x over previous
import dataclasses
import functools

import jax
import jax.numpy as jnp
from jax import lax
from jax.experimental import pallas as pl
from jax.experimental.pallas import tpu as pltpu
from jax.experimental.pallas import tpu_sc as plsc

def _sc_compiler_params():
    cp = pltpu.CompilerParams()
    if "needs_layout_passes" in pltpu.CompilerParams.__dataclass_fields__:
        cp = dataclasses.replace(cp, needs_layout_passes=False)
    return cp


NC = 2
NS = 16
NW = NC * NS
K = 112


def _sc_aggregate(y, ei_flat):
    n, d = y.shape
    e = ei_flat.shape[0] // 2
    e_per_w = e // NW
    chunks = e_per_w // K
    kt = e_per_w - chunks * K
    assert chunks % 4 == 1 and chunks >= 5
    assert 0 < kt <= 128 and kt % 8 == 0
    quads = (chunks - 1) // 4
    rps = (n // NS) // 8 * 8
    rlast = n - (NS - 1) * rps

    mesh = plsc.VectorSubcoreMesh(core_axis_name="c", subcore_axis_name="s")

    @functools.partial(
        pl.kernel,
        out_type=jax.ShapeDtypeStruct((NC, n, d), jnp.float32),
        mesh=mesh,
        scratch_types=[
            pltpu.VMEM((4, K), jnp.int32),
            pltpu.VMEM((4, K), jnp.int32),
            pltpu.VMEM((2, K, d), jnp.float32),
            pltpu.VMEM((kt,), jnp.int32),
            pltpu.VMEM((kt,), jnp.int32),
            pltpu.VMEM((kt, d), jnp.float32),
            pltpu.VMEM_SHARED((n, d), jnp.float32),
            pltpu.SemaphoreType.DMA,
            pltpu.SemaphoreType.DMA,
            pltpu.SemaphoreType.DMA,
            pltpu.SemaphoreType.DMA,
            pltpu.SemaphoreType.DMA,
            pltpu.SemaphoreType.DMA,
            pltpu.SemaphoreType.DMA,
            pltpu.SemaphoreType.DMA,
            pltpu.SemaphoreType.DMA,
            pltpu.SemaphoreType.DMA,
        ],
    )
    def k(y_hbm, ei_hbm, out_hbm, src_v, dst_v, rows_v, src_t, dst_t, rows_t,
          z_sh, si0, si1, si2, si3, sg0, sg1, ss0, ss1, sit, sgt):
        cid = lax.axis_index("c")
        sid = lax.axis_index("s")
        wid = cid * NS + sid
        r0 = sid * rps
        base = wid * e_per_w
        s_i = (si0, si1, si2, si3)
        s_g = (sg0, sg1)
        s_s = (ss0, ss1)

        def start_idx(c, q):
            off = base + c * K
            pltpu.async_copy(ei_hbm.at[pl.ds(off, K)], src_v.at[q], s_i[q])
            pltpu.async_copy(ei_hbm.at[pl.ds(e + off, K)], dst_v.at[q],
                             s_i[q])

        def wait_idx(c, q):
            off = base + c * K
            pltpu.make_async_copy(ei_hbm.at[pl.ds(off, K)], src_v.at[q],
                                  s_i[q]).wait()
            pltpu.make_async_copy(ei_hbm.at[pl.ds(e + off, K)], dst_v.at[q],
                                  s_i[q]).wait()

        def start_gather(q, b):
            pltpu.async_copy(y_hbm.at[src_v.at[q]], rows_v.at[b], s_g[b])

        def wait_gather(q, b):
            pltpu.make_async_copy(y_hbm.at[src_v.at[q]], rows_v.at[b],
                                  s_g[b]).wait()

        def start_scatter(q, b):
            pltpu.async_copy(rows_v.at[b], z_sh.at[dst_v.at[q]], s_s[b],
                             add=True)

        def wait_scatter(q, b):
            pltpu.make_async_copy(rows_v.at[b], z_sh.at[dst_v.at[q]],
                                  s_s[b]).wait()

        for q in range(4):
            start_idx(q, q)
        toff = base + chunks * K
        pltpu.async_copy(ei_hbm.at[pl.ds(toff, kt)], src_t, sit)
        pltpu.async_copy(ei_hbm.at[pl.ds(e + toff, kt)], dst_t, sit)

        @pl.when(sid < NS - 1)
        def _():
            pltpu.sync_copy(y_hbm.at[pl.ds(r0, rps)],
                            z_sh.at[pl.ds(r0, rps)])

        @pl.when(sid == NS - 1)
        def _():
            pltpu.sync_copy(y_hbm.at[pl.ds((NS - 1) * rps, rlast)],
                            z_sh.at[pl.ds((NS - 1) * rps, rlast)])

        wait_idx(0, 0)
        start_gather(0, 0)
        pltpu.make_async_copy(ei_hbm.at[pl.ds(toff, kt)], src_t, sit).wait()
        pltpu.make_async_copy(ei_hbm.at[pl.ds(e + toff, kt)], dst_t,
                              sit).wait()
        pltpu.async_copy(y_hbm.at[src_t], rows_t, sgt)
        plsc.subcore_barrier()

        @pl.loop(0, quads)
        def _(j):
            for q in range(4):
                c = 4 * j + q
                b = q % 2
                nb = 1 - b
                nq = (q + 1) % 4
                pq = (q + 3) % 4
                if q == 0:
                    @pl.when(j > 0)
                    def _():
                        wait_scatter(pq, nb)
                        start_idx(c + 3, pq)
                else:
                    wait_scatter(pq, nb)
                    if q <= 1:
                        start_idx(c + 3, pq)
                    else:
                        @pl.when(j < quads - 1)
                        def _():
                            start_idx(c + 3, pq)
                wait_idx(c + 1, nq)
                start_gather(nq, nb)
                wait_gather(q, b)
                start_scatter(q, b)

        wait_scatter(3, 1)
        wait_gather(0, 0)
        pltpu.sync_copy(rows_v.at[0], z_sh.at[dst_v.at[0]], add=True)
        pltpu.make_async_copy(y_hbm.at[src_t], rows_t, sgt).wait()
        pltpu.sync_copy(rows_t, z_sh.at[dst_t], add=True)

        plsc.subcore_barrier()

        @pl.when(sid < NS - 1)
        def _():
            pltpu.sync_copy(z_sh.at[pl.ds(r0, rps)],
                            out_hbm.at[cid, pl.ds(r0, rps)])

        @pl.when(sid == NS - 1)
        def _():
            pltpu.sync_copy(z_sh.at[pl.ds((NS - 1) * rps, rlast)],
                            out_hbm.at[cid, pl.ds((NS - 1) * rps, rlast)])

    return k(y, ei_flat)


def _sc_degree(ei_flat, n):
    e = ei_flat.shape[0] // 2
    e_per_w = e // NW

    mesh = plsc.VectorSubcoreMesh(core_axis_name="c", subcore_axis_name="s")

    @functools.partial(
        pl.kernel,
        out_type=jax.ShapeDtypeStruct((NW * n,), jnp.float32),
        mesh=mesh,
        scratch_types=[
            pltpu.VMEM((e_per_w,), jnp.int32),
            pltpu.VMEM((n,), jnp.float32),
        ],
        compiler_params=_sc_compiler_params(),
    )
    def k(ei_hbm, out_hbm, dst_v, deg_v):
        cid = lax.axis_index("c")
        sid = lax.axis_index("s")
        wid = cid * NS + sid
        pltpu.sync_copy(ei_hbm.at[pl.ds(e + wid * e_per_w, e_per_w)], dst_v)

        zeros = jnp.zeros((16,), jnp.float32)
        n_main = n // 128 * 128

        @pl.loop(0, n_main, step=128)
        def _(i):
            for u in range(8):
                deg_v[pl.ds(i + u * 16, 16)] = zeros

        for t in range(n_main, n, 16):
            deg_v[pl.ds(t, 16)] = zeros

        ones = jnp.ones((16,), jnp.float32)
        e_main = e_per_w // 128 * 128

        @pl.loop(0, e_main, step=128)
        def _(i):
            for u in range(8):
                plsc.addupdate_scatter(deg_v, [dst_v[pl.ds(i + u * 16, 16)]],
                                       ones)

        for t in range(e_main, e_per_w, 16):
            plsc.addupdate_scatter(deg_v, [dst_v[pl.ds(t, 16)]], ones)

        pltpu.sync_copy(deg_v, out_hbm.at[pl.ds(wid * n, n)])

    return k(ei_flat)


def _prescale_body(degp_ref, x_ref, dinv_ref, y1_ref):
    deg = jnp.sum(degp_ref[...], axis=1, keepdims=True) + 1.0
    dinv = lax.rsqrt(deg)
    dinv_ref[...] = dinv
    y1_ref[...] = x_ref[...] * dinv


def _tc_prescale(degp, x):
    n = x.shape[0]
    bn = 2000
    g = n // bn
    return pl.pallas_call(
        _prescale_body,
        grid=(g,),
        in_specs=[
            pl.BlockSpec((bn, NW), lambda i: (i, 0)),
            pl.BlockSpec((bn, 128), lambda i: (i, 0)),
        ],
        out_specs=[
            pl.BlockSpec((bn, 1), lambda i: (i, 0)),
            pl.BlockSpec((bn, 128), lambda i: (i, 0)),
        ],
        out_shape=[
            jax.ShapeDtypeStruct((n, 1), jnp.float32),
            jax.ShapeDtypeStruct((n, 128), jnp.float32),
        ],
    )(degp, x)


def _mid_body(z1p_ref, y1_ref, dinv_ref, w1_ref, b1_ref, w2_ref, y2_ref):
    dinv = dinv_ref[...]
    a = (z1p_ref[0] + z1p_ref[1] - y1_ref[...]) * dinv
    pre = jnp.dot(a.astype(jnp.bfloat16), w1_ref[...].astype(jnp.bfloat16),
                  preferred_element_type=jnp.float32)
    pre = pre + b1_ref[...][None, :]
    h = jnp.where(pre > 0, pre, 0.01 * pre)
    hw = jnp.dot(h.astype(jnp.bfloat16), w2_ref[...].astype(jnp.bfloat16),
                 preferred_element_type=jnp.float32)
    y2_ref[...] = hw * dinv


def _tc_mid(z1p, y1, dinv, W1, b1, W2):
    n = y1.shape[0]
    bn = 2000
    g = n // bn
    dh = W1.shape[1]
    return pl.pallas_call(
        _mid_body,
        grid=(g,),
        in_specs=[
            pl.BlockSpec((NC, bn, 128), lambda i: (0, i, 0)),
            pl.BlockSpec((bn, 128), lambda i: (i, 0)),
            pl.BlockSpec((bn, 1), lambda i: (i, 0)),
            pl.BlockSpec((128, dh), lambda i: (0, 0)),
            pl.BlockSpec((dh,), lambda i: (0,)),
            pl.BlockSpec((dh, 128), lambda i: (0, 0)),
        ],
        out_specs=pl.BlockSpec((bn, 128), lambda i: (i, 0)),
        out_shape=jax.ShapeDtypeStruct((n, 128), jnp.float32),
    )(z1p, y1, dinv, W1, b1, W2)


def _final_body(z2p_ref, y2_ref, dinv_ref, b2_ref, batch_ref, out_ref,
                sums_ref, cnt_ref, ng):
    i = pl.program_id(0)
    bn = y2_ref.shape[0]
    ne = (z2p_ref[0] + z2p_ref[1] - y2_ref[...]) * dinv_ref[...]
    ne = ne + b2_ref[...][None, :]
    seg = batch_ref[0]
    oh = (seg == lax.broadcasted_iota(jnp.int32, (ng, bn), 0))
    oh = oh.astype(jnp.float32)
    dn = (((1,), (0,)), ((), ()))
    sums = lax.dot_general(oh, ne, dn, preferred_element_type=jnp.float32)
    cnt = lax.dot_general(oh, jnp.ones((bn, 1), jnp.float32), dn,
                          preferred_element_type=jnp.float32)

    @pl.when(i == 0)
    def _():
        sums_ref[...] = jnp.zeros_like(sums_ref)
        cnt_ref[...] = jnp.zeros_like(cnt_ref)

    sums_ref[...] += sums
    cnt_ref[...] += cnt

    @pl.when(i == pl.num_programs(0) - 1)
    def _():
        out_ref[...] = sums_ref[...] / jnp.maximum(cnt_ref[...], 1.0)


def _tc_final(z2p, y2, dinv, b2, batch_rows, ng):
    n = y2.shape[0]
    bn = 2000
    g = n // bn
    return pl.pallas_call(
        functools.partial(_final_body, ng=ng),
        grid=(g,),
        in_specs=[
            pl.BlockSpec((NC, bn, 128), lambda i: (0, i, 0)),
            pl.BlockSpec((bn, 128), lambda i: (i, 0)),
            pl.BlockSpec((bn, 1), lambda i: (i, 0)),
            pl.BlockSpec((128,), lambda i: (0,)),
            pl.BlockSpec((1, 1, bn), lambda i: (i, 0, 0)),
        ],
        out_specs=pl.BlockSpec((ng, 128), lambda i: (0, 0)),
        out_shape=jax.ShapeDtypeStruct((ng, 128), jnp.float32),
        scratch_shapes=[
            pltpu.VMEM((ng, 128), jnp.float32),
            pltpu.VMEM((ng, 1), jnp.float32),
        ],
    )(z2p, y2, dinv, b2, batch_rows)


def kernel(x, edge_index, batch, W1, b1, W2, b2):
    n = x.shape[0]
    ng = 64
    ei_flat = edge_index.astype(jnp.int32).reshape(-1)
    batch_rows = batch.astype(jnp.int32).reshape(n // 2000, 1, 2000)

    degp = _sc_degree(ei_flat, n)
    degp = degp.reshape(NW, n).T
    dinv, y1 = _tc_prescale(degp, x)
    z1p = _sc_aggregate(y1, ei_flat)
    y2 = _tc_mid(z1p, y1, dinv, W1, b1, W2)
    z2p = _sc_aggregate(y2, ei_flat)
    return _tc_final(z2p, y2, dinv, b2, batch_rows, ng)
